# baseline (speedup 1.0000x reference)
import functools
import jax
import jax.numpy as jnp
from jax import lax
from jax.experimental import pallas as pl
from jax.experimental.pallas import tpu as pltpu
from jax.experimental.pallas import tpu_sc as plsc

VOCAB = 100000
EMBED_DIM = 64
BATCH = 16384

_NC = 2
_NS = 16
_NW = _NC * _NS
_B_PER_W = BATCH // _NW
_CHUNK = 128
_N_CHUNKS = _B_PER_W // _CHUNK
_L = 16


@functools.partial(
    pl.kernel,
    out_type=jax.ShapeDtypeStruct((EMBED_DIM, BATCH), jnp.float32),
    mesh=plsc.VectorSubcoreMesh(core_axis_name="c", subcore_axis_name="s"),
    scratch_types=[
        pltpu.VMEM((_B_PER_W,), jnp.int32),
        pltpu.VMEM((_B_PER_W,), jnp.int32),
        pltpu.VMEM((_B_PER_W,), jnp.int32),
        pltpu.VMEM((2, _CHUNK, 128), jnp.float32),
        pltpu.VMEM((EMBED_DIM, _CHUNK), jnp.float32),
        pltpu.SemaphoreType.DMA,
    ],
    compiler_params=pltpu.CompilerParams(needs_layout_passes=False),
)
def _gather_rows(idx_hbm, table_hbm, out_hbm, idx_v, v_v, s_v, blocks_v,
                 ot_v, gsem):
    wid = lax.axis_index("s") * _NC + lax.axis_index("c")
    base = wid * _B_PER_W

    pltpu.sync_copy(idx_hbm.at[wid], idx_v)

    for k in range(_B_PER_W // _L):
        ivec = idx_v[pl.ds(k * _L, _L)]
        v_v[pl.ds(k * _L, _L)] = lax.shift_right_logical(ivec, 1)
        s_v[pl.ds(k * _L, _L)] = lax.shift_left(
            lax.bitwise_and(ivec, 1), 6
        )

    iota = lax.iota(jnp.int32, _L)

    def fire(c, buf):
        return pltpu.async_copy(
            table_hbm.at[v_v.at[pl.ds(c * _CHUNK, _CHUNK)]],
            blocks_v.at[buf],
            gsem,
        )

    def extract(c, buf):
        for lg in range(_CHUNK // _L):
            l0 = lg * _L
            i0 = iota + l0
            s64 = s_v[pl.ds(c * _CHUNK + l0, _L)]

            def body(j, _):
                for u in range(4):
                    d = j * 4 + u
                    x = plsc.load_gather(blocks_v.at[buf], [i0, s64 + d])
                    ot_v[d, pl.ds(l0, _L)] = x
                return 0

            lax.fori_loop(0, EMBED_DIM // 4, body, 0)

    cp = fire(0, 0)
    for c in range(_N_CHUNKS):
        nxt = fire(c + 1, (c + 1) % 2) if c + 1 < _N_CHUNKS else None
        cp.wait()
        extract(c, c % 2)
        pltpu.sync_copy(
            ot_v, out_hbm.at[:, pl.ds(base + c * _CHUNK, _CHUNK)]
        )
        cp = nxt


def kernel(inputs, W):
    idx = inputs.reshape(_NW, _B_PER_W)
    table = W.reshape(VOCAB // 2, 2 * EMBED_DIM)
    return _gather_rows(idx, table).T
